# Initial kernel scaffold; baseline (speedup 1.0000x reference)
import jax
import jax.numpy as jnp
from jax.experimental import pallas as pl


def kernel(x, edge_index, batch, W_enc, b_enc, Wc0, bc0, Wc1, bc1, Wc2, bc2, Wc3, bc3, W_node, b_node, Wg1, bg1, Wg2, bg2):
    raise NotImplementedError("write your pallas kernel here")


# speedup vs baseline: 12.1437x; 12.1437x over previous
import functools

import jax
import jax.numpy as jnp
from jax import lax
from jax.experimental import pallas as pl
from jax.experimental.pallas import tpu as pltpu
from jax.experimental.pallas import tpu_sc as plsc

N = 50000
E = 800000
G = 16
D_IN = 128
D_H = 64
D_NODE = 6
D_GLOBAL = 4

DUMP = 176
TROWS = N + DUMP
EPAD = 819200
CHUNK = 640
ZROWS = TROWS // 16
OROWS = 3128

def _sc_mesh():
    return plsc.VectorSubcoreMesh(core_axis_name="c", subcore_axis_name="s",
                                  num_cores=2, num_subcores=16)


@functools.cache
def _deg_kernel():
    @functools.partial(
        pl.kernel,
        mesh=_sc_mesh(),
        compiler_params=pltpu.CompilerParams(use_tc_tiling_on_sc=False),
        out_type=jax.ShapeDtypeStruct((2, N, 8), jnp.float32),
        scratch_types=[
            pltpu.VMEM((1024,), jnp.int32),
            pltpu.VMEM((1024, 8), jnp.float32),
            pltpu.VMEM_SHARED((TROWS, 8), jnp.float32),
        ],
    )
    def deg(dst_hbm, ones_hbm, z8_hbm, out_hbm, didx, ones_v, table):
        c = lax.axis_index("c")
        s = lax.axis_index("s")
        pltpu.sync_copy(z8_hbm, table.at[pl.ds(s * ZROWS, ZROWS)])
        pltpu.sync_copy(ones_hbm, ones_v)
        plsc.subcore_barrier()

        def chunk(k, carry):
            eb = c * (EPAD // 2) + s * (EPAD // 32) + k * 1024
            pltpu.sync_copy(dst_hbm.at[pl.ds(eb, 1024)], didx)
            pltpu.sync_copy(ones_v, table.at[didx], add=True)
            return carry

        lax.fori_loop(0, 25, chunk, 0)
        plsc.subcore_barrier()
        off = pl.multiple_of(jnp.minimum(s * OROWS, N - OROWS), 8)
        pltpu.sync_copy(table.at[pl.ds(off, OROWS)],
                        out_hbm.at[c, pl.ds(off, OROWS)])

    return deg


@functools.cache
def _scatter_kernel():
    @functools.partial(
        pl.kernel,
        mesh=_sc_mesh(),
        compiler_params=pltpu.CompilerParams(use_tc_tiling_on_sc=False),
        out_type=jax.ShapeDtypeStruct((2, N, 32), jnp.float32),
        scratch_types=[
            pltpu.VMEM((CHUNK,), jnp.int32),
            pltpu.VMEM((CHUNK,), jnp.int32),
            pltpu.VMEM((CHUNK, 32), jnp.float32),
            pltpu.VMEM_SHARED((TROWS, 32), jnp.float32),
            pltpu.SemaphoreType.DMA,
        ],
    )
    def scatter(u_hbm, srcb_hbm, dst_hbm, z32_hbm, out_hbm,
                sidx, didx, rows, table, sem):
        c = lax.axis_index("c")
        s = lax.axis_index("s")
        pltpu.sync_copy(z32_hbm, table.at[pl.ds(s * ZROWS, ZROWS)])
        plsc.subcore_barrier()

        def chunk(k, carry):
            eb = s * (EPAD // 16) + k * CHUNK
            pltpu.sync_copy(srcb_hbm.at[pl.ds(c * EPAD + eb, CHUNK)], sidx)
            pltpu.sync_copy(dst_hbm.at[pl.ds(eb, CHUNK)], didx)
            pltpu.async_copy(u_hbm.at[sidx], rows, sem).wait()
            pltpu.sync_copy(rows, table.at[didx], add=True)
            return carry

        lax.fori_loop(0, (EPAD // 16) // CHUNK, chunk, 0)
        plsc.subcore_barrier()
        off = pl.multiple_of(jnp.minimum(s * OROWS, N - OROWS), 8)
        pltpu.sync_copy(table.at[pl.ds(off, OROWS)],
                        out_hbm.at[c, pl.ds(off, OROWS)])

    return scatter


BLK = 400
GRID = N // BLK


def _prep_body(x_ref, dp_ref, we_ref, be_ref, w0_ref, u_ref, dis_ref):
    deg = 1.0 + dp_ref[0][:, 0:1] + dp_ref[1][:, 0:1]
    dis = lax.rsqrt(deg)
    h = jnp.maximum(
        jnp.dot(x_ref[...], we_ref[...], preferred_element_type=jnp.float32)
        + be_ref[...], 0.0)
    un = dis * jnp.dot(h, w0_ref[...], preferred_element_type=jnp.float32)
    u_ref[0] = un[:, :32]
    u_ref[1] = un[:, 32:]
    dis_ref[...] = dis


def _tc_prep(x, degp, W_enc, be, Wc0):
    return pl.pallas_call(
        _prep_body,
        grid=(GRID,),
        in_specs=[
            pl.BlockSpec((BLK, D_IN), lambda i: (i, 0)),
            pl.BlockSpec((2, BLK, 8), lambda i: (0, i, 0)),
            pl.BlockSpec((D_IN, D_H), lambda i: (0, 0)),
            pl.BlockSpec((1, D_H), lambda i: (0, 0)),
            pl.BlockSpec((D_H, D_H), lambda i: (0, 0)),
        ],
        out_specs=[
            pl.BlockSpec((2, BLK, 32), lambda i: (0, i, 0)),
            pl.BlockSpec((BLK, 1), lambda i: (i, 0)),
        ],
        out_shape=[
            jax.ShapeDtypeStruct((2, N, 32), jnp.float32),
            jax.ShapeDtypeStruct((N, 1), jnp.float32),
        ],
    )(x, degp, W_enc, be, Wc0)


def _layer_body(a_ref, u_ref, dis_ref, b_ref, w_ref, un_ref):
    su = jnp.concatenate([a_ref[0] + u_ref[0], a_ref[1] + u_ref[1]], axis=1)
    dis = dis_ref[...]
    h = jnp.maximum(dis * su + b_ref[...], 0.0)
    un = dis * jnp.dot(h, w_ref[...], preferred_element_type=jnp.float32)
    un_ref[0] = un[:, :32]
    un_ref[1] = un[:, 32:]


def _tc_layer(a, u, dis, b_prev, W_next):
    return pl.pallas_call(
        _layer_body,
        grid=(GRID,),
        in_specs=[
            pl.BlockSpec((2, BLK, 32), lambda i: (0, i, 0)),
            pl.BlockSpec((2, BLK, 32), lambda i: (0, i, 0)),
            pl.BlockSpec((BLK, 1), lambda i: (i, 0)),
            pl.BlockSpec((1, D_H), lambda i: (0, 0)),
            pl.BlockSpec((D_H, D_H), lambda i: (0, 0)),
        ],
        out_specs=pl.BlockSpec((2, BLK, 32), lambda i: (0, i, 0)),
        out_shape=jax.ShapeDtypeStruct((2, N, 32), jnp.float32),
    )(a, u, dis, b_prev, W_next)


def _final_body(a_ref, u_ref, dis_ref, b_ref, wn_ref, bn_ref, batch_ref,
                wg1_ref, bg1_ref, wg2_ref, bg2_ref, pn_ref, pg_ref,
                sums, counts):
    i = pl.program_id(0)
    su = jnp.concatenate([a_ref[0] + u_ref[0], a_ref[1] + u_ref[1]], axis=1)
    dis = dis_ref[...]
    h = jnp.maximum(dis * su + b_ref[...], 0.0)
    pn_ref[...] = (jnp.dot(h, wn_ref[...], preferred_element_type=jnp.float32)
                   + bn_ref[...])
    onehot = (batch_ref[...] ==
              lax.broadcasted_iota(jnp.int32, (BLK, G), 1)).astype(jnp.float32)
    s_inc = lax.dot_general(onehot, h, (((0,), (0,)), ((), ())),
                            preferred_element_type=jnp.float32)
    c_inc = lax.dot_general(onehot, jnp.ones((BLK, 1), jnp.float32),
                            (((0,), (0,)), ((), ())),
                            preferred_element_type=jnp.float32)

    @pl.when(i == 0)
    def _():
        sums[...] = jnp.zeros_like(sums)
        counts[...] = jnp.zeros_like(counts)

    sums[...] += s_inc
    counts[...] += c_inc

    @pl.when(i == GRID - 1)
    def _():
        emb = sums[...] / jnp.maximum(counts[...], 1.0)
        g = jnp.maximum(
            jnp.dot(emb, wg1_ref[...], preferred_element_type=jnp.float32)
            + bg1_ref[...], 0.0)
        pg_ref[...] = (jnp.dot(g, wg2_ref[...],
                               preferred_element_type=jnp.float32)
                       + bg2_ref[...])


def _tc_final(a, u, dis, b3, W_node, bn, batch2d, Wg1, bg1r, Wg2, bg2r):
    return pl.pallas_call(
        _final_body,
        grid=(GRID,),
        in_specs=[
            pl.BlockSpec((2, BLK, 32), lambda i: (0, i, 0)),
            pl.BlockSpec((2, BLK, 32), lambda i: (0, i, 0)),
            pl.BlockSpec((BLK, 1), lambda i: (i, 0)),
            pl.BlockSpec((1, D_H), lambda i: (0, 0)),
            pl.BlockSpec((D_H, D_NODE), lambda i: (0, 0)),
            pl.BlockSpec((1, D_NODE), lambda i: (0, 0)),
            pl.BlockSpec((BLK, 1), lambda i: (i, 0)),
            pl.BlockSpec((D_H, D_H), lambda i: (0, 0)),
            pl.BlockSpec((1, D_H), lambda i: (0, 0)),
            pl.BlockSpec((D_H, D_GLOBAL), lambda i: (0, 0)),
            pl.BlockSpec((1, D_GLOBAL), lambda i: (0, 0)),
        ],
        out_specs=[
            pl.BlockSpec((BLK, D_NODE), lambda i: (i, 0)),
            pl.BlockSpec((G, D_GLOBAL), lambda i: (0, 0)),
        ],
        out_shape=[
            jax.ShapeDtypeStruct((N, D_NODE), jnp.float32),
            jax.ShapeDtypeStruct((G, D_GLOBAL), jnp.float32),
        ],
        scratch_shapes=[
            pltpu.VMEM((G, D_H), jnp.float32),
            pltpu.VMEM((G, 1), jnp.float32),
        ],
    )(a, u, dis, b3, W_node, bn, batch2d, Wg1, bg1r, Wg2, bg2r)


def kernel(x, edge_index, batch, W_enc, b_enc, Wc0, bc0, Wc1, bc1, Wc2, bc2,
           Wc3, bc3, W_node, b_node, Wg1, bg1, Wg2, bg2):
    f32 = jnp.float32
    src = edge_index[0]
    dst = edge_index[1]
    pad = EPAD - E
    srcp = jnp.concatenate([src, jnp.zeros((pad,), jnp.int32)])
    dstp = jnp.concatenate(
        [dst, N + (jnp.arange(pad, dtype=jnp.int32) % DUMP)])
    src_both = jnp.concatenate([srcp, srcp + N])
    dst1d = dstp
    ones8 = jnp.ones((1024, 8), f32)
    z8 = jnp.zeros((ZROWS, 8), f32)
    z32 = jnp.zeros((ZROWS, 32), f32)
    batch2d = batch.reshape(N, 1)
    be = b_enc.reshape(1, D_H)
    b0 = bc0.reshape(1, D_H)
    b1 = bc1.reshape(1, D_H)
    b2 = bc2.reshape(1, D_H)
    b3 = bc3.reshape(1, D_H)
    bn = b_node.reshape(1, D_NODE)
    bg1r = bg1.reshape(1, D_H)
    bg2r = bg2.reshape(1, D_GLOBAL)

    degp = _deg_kernel()(dst1d, ones8, z8)
    u, dis = _tc_prep(x, degp, W_enc, be, Wc0)
    a = _scatter_kernel()(u.reshape(2 * N, 32), src_both, dst1d, z32)
    u = _tc_layer(a, u, dis, b0, Wc1)
    a = _scatter_kernel()(u.reshape(2 * N, 32), src_both, dst1d, z32)
    u = _tc_layer(a, u, dis, b1, Wc2)
    a = _scatter_kernel()(u.reshape(2 * N, 32), src_both, dst1d, z32)
    u = _tc_layer(a, u, dis, b2, Wc3)
    a = _scatter_kernel()(u.reshape(2 * N, 32), src_both, dst1d, z32)
    pred_node, pred_global = _tc_final(a, u, dis, b3, W_node, bn, batch2d,
                                       Wg1, bg1r, Wg2, bg2r)
    return (pred_node, pred_global)


# speedup vs baseline: 13.8242x; 1.1384x over previous
import functools

import jax
import jax.numpy as jnp
from jax import lax
from jax.experimental import pallas as pl
from jax.experimental.pallas import tpu as pltpu
from jax.experimental.pallas import tpu_sc as plsc

N = 50000
E = 800000
G = 16
D_IN = 128
D_H = 64
D_NODE = 6
D_GLOBAL = 4

DUMP = 176
TROWS = N + DUMP
EPAD = 819200
CHUNK = 400
ZROWS = TROWS // 16
OROWS = 3128

def _sc_mesh():
    return plsc.VectorSubcoreMesh(core_axis_name="c", subcore_axis_name="s",
                                  num_cores=2, num_subcores=16)


@functools.cache
def _deg_kernel():
    @functools.partial(
        pl.kernel,
        mesh=_sc_mesh(),
        compiler_params=pltpu.CompilerParams(use_tc_tiling_on_sc=False),
        out_type=jax.ShapeDtypeStruct((2, N, 8), jnp.float32),
        scratch_types=[
            pltpu.VMEM((1024,), jnp.int32),
            pltpu.VMEM((1024, 8), jnp.float32),
            pltpu.VMEM_SHARED((TROWS, 8), jnp.float32),
        ],
    )
    def deg(dst_hbm, ones_hbm, z8_hbm, out_hbm, didx, ones_v, table):
        c = lax.axis_index("c")
        s = lax.axis_index("s")
        pltpu.sync_copy(z8_hbm, table.at[pl.ds(s * ZROWS, ZROWS)])
        pltpu.sync_copy(ones_hbm, ones_v)
        plsc.subcore_barrier()

        def chunk(k, carry):
            eb = c * (EPAD // 2) + s * (EPAD // 32) + k * 1024
            pltpu.sync_copy(dst_hbm.at[pl.ds(eb, 1024)], didx)
            pltpu.sync_copy(ones_v, table.at[didx], add=True)
            return carry

        lax.fori_loop(0, 25, chunk, 0)
        plsc.subcore_barrier()
        off = pl.multiple_of(jnp.minimum(s * OROWS, N - OROWS), 8)
        pltpu.sync_copy(table.at[pl.ds(off, OROWS)],
                        out_hbm.at[c, pl.ds(off, OROWS)])

    return deg


@functools.cache
def _scatter_kernel():
    npair = (EPAD // 16) // CHUNK // 2

    @functools.partial(
        pl.kernel,
        mesh=_sc_mesh(),
        compiler_params=pltpu.CompilerParams(use_tc_tiling_on_sc=False),
        out_type=jax.ShapeDtypeStruct((2, N, 32), jnp.float32),
        scratch_types=[
            pltpu.VMEM((CHUNK,), jnp.int32),
            pltpu.VMEM((CHUNK,), jnp.int32),
            pltpu.VMEM((CHUNK,), jnp.int32),
            pltpu.VMEM((CHUNK,), jnp.int32),
            pltpu.VMEM((CHUNK, 32), jnp.float32),
            pltpu.VMEM((CHUNK, 32), jnp.float32),
            pltpu.VMEM_SHARED((TROWS, 32), jnp.float32),
            pltpu.SemaphoreType.DMA,
            pltpu.SemaphoreType.DMA,
        ],
    )
    def scatter(u_hbm, srcb_hbm, dst_hbm, z32_hbm, out_hbm,
                sidx0, didx0, sidx1, didx1, rows0, rows1, table, sem0, sem1):
        c = lax.axis_index("c")
        s = lax.axis_index("s")
        base = s * (EPAD // 16)
        pltpu.sync_copy(z32_hbm, table.at[pl.ds(s * ZROWS, ZROWS)])
        plsc.subcore_barrier()

        pltpu.sync_copy(srcb_hbm.at[pl.ds(c * EPAD + base, CHUNK)], sidx0)
        pltpu.sync_copy(dst_hbm.at[pl.ds(base, CHUNK)], didx0)
        pltpu.async_copy(u_hbm.at[sidx0], rows0, sem0)

        def pair(j, carry):
            e1 = base + (2 * j + 1) * CHUNK
            pltpu.sync_copy(srcb_hbm.at[pl.ds(c * EPAD + e1, CHUNK)], sidx1)
            pltpu.sync_copy(dst_hbm.at[pl.ds(e1, CHUNK)], didx1)
            pltpu.async_copy(u_hbm.at[sidx1], rows1, sem1)
            pltpu.make_async_copy(u_hbm.at[sidx0], rows0, sem0).wait()
            pltpu.sync_copy(rows0, table.at[didx0], add=True)

            @pl.when(j < npair - 1)
            def _():
                e2 = base + (2 * j + 2) * CHUNK
                pltpu.sync_copy(srcb_hbm.at[pl.ds(c * EPAD + e2, CHUNK)],
                                sidx0)
                pltpu.sync_copy(dst_hbm.at[pl.ds(e2, CHUNK)], didx0)
                pltpu.async_copy(u_hbm.at[sidx0], rows0, sem0)

            pltpu.make_async_copy(u_hbm.at[sidx1], rows1, sem1).wait()
            pltpu.sync_copy(rows1, table.at[didx1], add=True)
            return carry

        lax.fori_loop(0, npair, pair, 0)
        plsc.subcore_barrier()
        off = pl.multiple_of(jnp.minimum(s * OROWS, N - OROWS), 8)
        pltpu.sync_copy(table.at[pl.ds(off, OROWS)],
                        out_hbm.at[c, pl.ds(off, OROWS)])

    return scatter


BLK = 400
GRID = N // BLK


def _prep_body(x_ref, dp_ref, we_ref, be_ref, w0_ref, u_ref, dis_ref):
    deg = 1.0 + dp_ref[0][:, 0:1] + dp_ref[1][:, 0:1]
    dis = lax.rsqrt(deg)
    h = jnp.maximum(
        jnp.dot(x_ref[...], we_ref[...], preferred_element_type=jnp.float32)
        + be_ref[...], 0.0)
    un = dis * jnp.dot(h, w0_ref[...], preferred_element_type=jnp.float32)
    u_ref[0] = un[:, :32]
    u_ref[1] = un[:, 32:]
    dis_ref[...] = dis


def _tc_prep(x, degp, W_enc, be, Wc0):
    return pl.pallas_call(
        _prep_body,
        grid=(GRID,),
        in_specs=[
            pl.BlockSpec((BLK, D_IN), lambda i: (i, 0)),
            pl.BlockSpec((2, BLK, 8), lambda i: (0, i, 0)),
            pl.BlockSpec((D_IN, D_H), lambda i: (0, 0)),
            pl.BlockSpec((1, D_H), lambda i: (0, 0)),
            pl.BlockSpec((D_H, D_H), lambda i: (0, 0)),
        ],
        out_specs=[
            pl.BlockSpec((2, BLK, 32), lambda i: (0, i, 0)),
            pl.BlockSpec((BLK, 1), lambda i: (i, 0)),
        ],
        out_shape=[
            jax.ShapeDtypeStruct((2, N, 32), jnp.float32),
            jax.ShapeDtypeStruct((N, 1), jnp.float32),
        ],
    )(x, degp, W_enc, be, Wc0)


def _layer_body(a_ref, u_ref, dis_ref, b_ref, w_ref, un_ref):
    su = jnp.concatenate([a_ref[0] + u_ref[0], a_ref[1] + u_ref[1]], axis=1)
    dis = dis_ref[...]
    h = jnp.maximum(dis * su + b_ref[...], 0.0)
    un = dis * jnp.dot(h, w_ref[...], preferred_element_type=jnp.float32)
    un_ref[0] = un[:, :32]
    un_ref[1] = un[:, 32:]


def _tc_layer(a, u, dis, b_prev, W_next):
    return pl.pallas_call(
        _layer_body,
        grid=(GRID,),
        in_specs=[
            pl.BlockSpec((2, BLK, 32), lambda i: (0, i, 0)),
            pl.BlockSpec((2, BLK, 32), lambda i: (0, i, 0)),
            pl.BlockSpec((BLK, 1), lambda i: (i, 0)),
            pl.BlockSpec((1, D_H), lambda i: (0, 0)),
            pl.BlockSpec((D_H, D_H), lambda i: (0, 0)),
        ],
        out_specs=pl.BlockSpec((2, BLK, 32), lambda i: (0, i, 0)),
        out_shape=jax.ShapeDtypeStruct((2, N, 32), jnp.float32),
    )(a, u, dis, b_prev, W_next)


def _final_body(a_ref, u_ref, dis_ref, b_ref, wn_ref, bn_ref, batch_ref,
                wg1_ref, bg1_ref, wg2_ref, bg2_ref, pn_ref, pg_ref,
                sums, counts):
    i = pl.program_id(0)
    su = jnp.concatenate([a_ref[0] + u_ref[0], a_ref[1] + u_ref[1]], axis=1)
    dis = dis_ref[...]
    h = jnp.maximum(dis * su + b_ref[...], 0.0)
    pn_ref[...] = (jnp.dot(h, wn_ref[...], preferred_element_type=jnp.float32)
                   + bn_ref[...])
    onehot = (batch_ref[...] ==
              lax.broadcasted_iota(jnp.int32, (BLK, G), 1)).astype(jnp.float32)
    s_inc = lax.dot_general(onehot, h, (((0,), (0,)), ((), ())),
                            preferred_element_type=jnp.float32)
    c_inc = lax.dot_general(onehot, jnp.ones((BLK, 1), jnp.float32),
                            (((0,), (0,)), ((), ())),
                            preferred_element_type=jnp.float32)

    @pl.when(i == 0)
    def _():
        sums[...] = jnp.zeros_like(sums)
        counts[...] = jnp.zeros_like(counts)

    sums[...] += s_inc
    counts[...] += c_inc

    @pl.when(i == GRID - 1)
    def _():
        emb = sums[...] / jnp.maximum(counts[...], 1.0)
        g = jnp.maximum(
            jnp.dot(emb, wg1_ref[...], preferred_element_type=jnp.float32)
            + bg1_ref[...], 0.0)
        pg_ref[...] = (jnp.dot(g, wg2_ref[...],
                               preferred_element_type=jnp.float32)
                       + bg2_ref[...])


def _tc_final(a, u, dis, b3, W_node, bn, batch2d, Wg1, bg1r, Wg2, bg2r):
    return pl.pallas_call(
        _final_body,
        grid=(GRID,),
        in_specs=[
            pl.BlockSpec((2, BLK, 32), lambda i: (0, i, 0)),
            pl.BlockSpec((2, BLK, 32), lambda i: (0, i, 0)),
            pl.BlockSpec((BLK, 1), lambda i: (i, 0)),
            pl.BlockSpec((1, D_H), lambda i: (0, 0)),
            pl.BlockSpec((D_H, D_NODE), lambda i: (0, 0)),
            pl.BlockSpec((1, D_NODE), lambda i: (0, 0)),
            pl.BlockSpec((BLK, 1), lambda i: (i, 0)),
            pl.BlockSpec((D_H, D_H), lambda i: (0, 0)),
            pl.BlockSpec((1, D_H), lambda i: (0, 0)),
            pl.BlockSpec((D_H, D_GLOBAL), lambda i: (0, 0)),
            pl.BlockSpec((1, D_GLOBAL), lambda i: (0, 0)),
        ],
        out_specs=[
            pl.BlockSpec((BLK, D_NODE), lambda i: (i, 0)),
            pl.BlockSpec((G, D_GLOBAL), lambda i: (0, 0)),
        ],
        out_shape=[
            jax.ShapeDtypeStruct((N, D_NODE), jnp.float32),
            jax.ShapeDtypeStruct((G, D_GLOBAL), jnp.float32),
        ],
        scratch_shapes=[
            pltpu.VMEM((G, D_H), jnp.float32),
            pltpu.VMEM((G, 1), jnp.float32),
        ],
    )(a, u, dis, b3, W_node, bn, batch2d, Wg1, bg1r, Wg2, bg2r)


def kernel(x, edge_index, batch, W_enc, b_enc, Wc0, bc0, Wc1, bc1, Wc2, bc2,
           Wc3, bc3, W_node, b_node, Wg1, bg1, Wg2, bg2):
    f32 = jnp.float32
    src = edge_index[0]
    dst = edge_index[1]
    pad = EPAD - E
    srcp = jnp.concatenate([src, jnp.zeros((pad,), jnp.int32)])
    dstp = jnp.concatenate(
        [dst, N + (jnp.arange(pad, dtype=jnp.int32) % DUMP)])
    src_both = jnp.concatenate([srcp, srcp + N])
    dst1d = dstp
    ones8 = jnp.ones((1024, 8), f32)
    z8 = jnp.zeros((ZROWS, 8), f32)
    z32 = jnp.zeros((ZROWS, 32), f32)
    batch2d = batch.reshape(N, 1)
    be = b_enc.reshape(1, D_H)
    b0 = bc0.reshape(1, D_H)
    b1 = bc1.reshape(1, D_H)
    b2 = bc2.reshape(1, D_H)
    b3 = bc3.reshape(1, D_H)
    bn = b_node.reshape(1, D_NODE)
    bg1r = bg1.reshape(1, D_H)
    bg2r = bg2.reshape(1, D_GLOBAL)

    degp = _deg_kernel()(dst1d, ones8, z8)
    u, dis = _tc_prep(x, degp, W_enc, be, Wc0)
    a = _scatter_kernel()(u.reshape(2 * N, 32), src_both, dst1d, z32)
    u = _tc_layer(a, u, dis, b0, Wc1)
    a = _scatter_kernel()(u.reshape(2 * N, 32), src_both, dst1d, z32)
    u = _tc_layer(a, u, dis, b1, Wc2)
    a = _scatter_kernel()(u.reshape(2 * N, 32), src_both, dst1d, z32)
    u = _tc_layer(a, u, dis, b2, Wc3)
    a = _scatter_kernel()(u.reshape(2 * N, 32), src_both, dst1d, z32)
    pred_node, pred_global = _tc_final(a, u, dis, b3, W_node, bn, batch2d,
                                       Wg1, bg1r, Wg2, bg2r)
    return (pred_node, pred_global)


# speedup vs baseline: 14.3207x; 1.0359x over previous
import functools

import jax
import jax.numpy as jnp
from jax import lax
from jax.experimental import pallas as pl
from jax.experimental.pallas import tpu as pltpu
from jax.experimental.pallas import tpu_sc as plsc

N = 50000
E = 800000
G = 16
D_IN = 128
D_H = 64
D_NODE = 6
D_GLOBAL = 4

DUMP = 176
TROWS = N + DUMP
EPAD = 819200
CHUNK = 320
GSZ = 8 * CHUNK
NITER = (EPAD // 16) // (8 * CHUNK)
ZROWS = TROWS // 16
OROWS = 3128

def _sc_mesh():
    return plsc.VectorSubcoreMesh(core_axis_name="c", subcore_axis_name="s",
                                  num_cores=2, num_subcores=16)


@functools.cache
def _deg_kernel():
    @functools.partial(
        pl.kernel,
        mesh=_sc_mesh(),
        compiler_params=pltpu.CompilerParams(use_tc_tiling_on_sc=False),
        out_type=jax.ShapeDtypeStruct((2, N, 8), jnp.float32),
        scratch_types=[
            pltpu.VMEM((1024,), jnp.int32),
            pltpu.VMEM((1024, 8), jnp.float32),
            pltpu.VMEM_SHARED((TROWS, 8), jnp.float32),
        ],
    )
    def deg(dst_hbm, ones_hbm, z8_hbm, out_hbm, didx, ones_v, table):
        c = lax.axis_index("c")
        s = lax.axis_index("s")
        pltpu.sync_copy(z8_hbm, table.at[pl.ds(s * ZROWS, ZROWS)])
        pltpu.sync_copy(ones_hbm, ones_v)
        plsc.subcore_barrier()

        def chunk(k, carry):
            eb = c * (EPAD // 2) + s * (EPAD // 32) + k * 1024
            pltpu.sync_copy(dst_hbm.at[pl.ds(eb, 1024)], didx)
            pltpu.sync_copy(ones_v, table.at[didx], add=True)
            return carry

        lax.fori_loop(0, 25, chunk, 0)
        plsc.subcore_barrier()
        off = pl.multiple_of(jnp.minimum(s * OROWS, N - OROWS), 8)
        pltpu.sync_copy(table.at[pl.ds(off, OROWS)],
                        out_hbm.at[c, pl.ds(off, OROWS)])

    return deg


@functools.cache
def _scatter_kernel():
    @functools.partial(
        pl.kernel,
        mesh=_sc_mesh(),
        compiler_params=pltpu.CompilerParams(use_tc_tiling_on_sc=False),
        out_type=jax.ShapeDtypeStruct((2, N, 32), jnp.float32),
        scratch_types=[
            pltpu.VMEM((GSZ,), jnp.int32),
            pltpu.VMEM((GSZ,), jnp.int32),
            pltpu.VMEM((CHUNK, 32), jnp.float32),
            pltpu.VMEM((CHUNK, 32), jnp.float32),
            pltpu.VMEM_SHARED((TROWS, 32), jnp.float32),
            pltpu.SemaphoreType.DMA,
            pltpu.SemaphoreType.DMA,
            pltpu.SemaphoreType.DMA,
            pltpu.SemaphoreType.DMA,
        ],
    )
    def scatter(u_hbm, sd_hbm, z32_hbm, out_hbm,
                sdA, sdB, rows0, rows1, table, semA, semB, sem0, sem1):
        c = lax.axis_index("c")
        s = lax.axis_index("s")
        sdbase = c * (2 * EPAD) + s * (2 * (EPAD // 16))
        rows = (rows0, rows1)
        gsem = (sem0, sem1)

        def sidx(buf, q):
            return buf.at[pl.ds(q * 2 * CHUNK, CHUNK)]

        def didx(buf, q):
            return buf.at[pl.ds(q * 2 * CHUNK + CHUNK, CHUNK)]

        pltpu.sync_copy(sd_hbm.at[pl.ds(sdbase, GSZ)], sdA)
        pltpu.async_copy(u_hbm.at[sidx(sdA, 0)], rows0, sem0)
        pltpu.sync_copy(z32_hbm, table.at[pl.ds(s * ZROWS, ZROWS)])
        plsc.subcore_barrier()

        def iter_body(t, carry):
            goff = sdbase + t * 2 * GSZ
            pltpu.async_copy(sd_hbm.at[pl.ds(goff + GSZ, GSZ)], sdB, semB)
            for q in range(4):
                slot, nslot = q % 2, (q + 1) % 2
                if q < 3:
                    pltpu.async_copy(u_hbm.at[sidx(sdA, q + 1)],
                                     rows[nslot], gsem[nslot])
                else:
                    pltpu.make_async_copy(
                        sd_hbm.at[pl.ds(goff + GSZ, GSZ)], sdB, semB).wait()
                    pltpu.async_copy(u_hbm.at[sidx(sdB, 0)],
                                     rows[nslot], gsem[nslot])
                pltpu.make_async_copy(u_hbm.at[sidx(sdA, q)],
                                      rows[slot], gsem[slot]).wait()
                pltpu.sync_copy(rows[slot], table.at[didx(sdA, q)], add=True)

            @pl.when(t < NITER - 1)
            def _():
                pltpu.async_copy(sd_hbm.at[pl.ds(goff + 2 * GSZ, GSZ)],
                                 sdA, semA)

            for q in range(4):
                slot, nslot = q % 2, (q + 1) % 2
                if q < 3:
                    pltpu.async_copy(u_hbm.at[sidx(sdB, q + 1)],
                                     rows[nslot], gsem[nslot])
                else:
                    @pl.when(t < NITER - 1)
                    def _():
                        pltpu.make_async_copy(
                            sd_hbm.at[pl.ds(goff + 2 * GSZ, GSZ)],
                            sdA, semA).wait()
                        pltpu.async_copy(u_hbm.at[sidx(sdA, 0)],
                                         rows[nslot], gsem[nslot])
                pltpu.make_async_copy(u_hbm.at[sidx(sdB, q)],
                                      rows[slot], gsem[slot]).wait()
                pltpu.sync_copy(rows[slot], table.at[didx(sdB, q)], add=True)
            return carry

        lax.fori_loop(0, NITER, iter_body, 0)
        plsc.subcore_barrier()
        off = pl.multiple_of(jnp.minimum(s * OROWS, N - OROWS), 8)
        pltpu.sync_copy(table.at[pl.ds(off, OROWS)],
                        out_hbm.at[c, pl.ds(off, OROWS)])

    return scatter


BLK = 400
GRID = N // BLK


def _prep_body(x_ref, dp_ref, we_ref, be_ref, w0_ref, u_ref, dis_ref):
    deg = 1.0 + dp_ref[0][:, 0:1] + dp_ref[1][:, 0:1]
    dis = lax.rsqrt(deg)
    h = jnp.maximum(
        jnp.dot(x_ref[...], we_ref[...], preferred_element_type=jnp.float32)
        + be_ref[...], 0.0)
    un = dis * jnp.dot(h, w0_ref[...], preferred_element_type=jnp.float32)
    u_ref[0] = un[:, :32]
    u_ref[1] = un[:, 32:]
    dis_ref[...] = dis


def _tc_prep(x, degp, W_enc, be, Wc0):
    return pl.pallas_call(
        _prep_body,
        grid=(GRID,),
        in_specs=[
            pl.BlockSpec((BLK, D_IN), lambda i: (i, 0)),
            pl.BlockSpec((2, BLK, 8), lambda i: (0, i, 0)),
            pl.BlockSpec((D_IN, D_H), lambda i: (0, 0)),
            pl.BlockSpec((1, D_H), lambda i: (0, 0)),
            pl.BlockSpec((D_H, D_H), lambda i: (0, 0)),
        ],
        out_specs=[
            pl.BlockSpec((2, BLK, 32), lambda i: (0, i, 0)),
            pl.BlockSpec((BLK, 1), lambda i: (i, 0)),
        ],
        out_shape=[
            jax.ShapeDtypeStruct((2, N, 32), jnp.float32),
            jax.ShapeDtypeStruct((N, 1), jnp.float32),
        ],
    )(x, degp, W_enc, be, Wc0)


def _layer_body(a_ref, u_ref, dis_ref, b_ref, w_ref, un_ref):
    su = jnp.concatenate([a_ref[0] + u_ref[0], a_ref[1] + u_ref[1]], axis=1)
    dis = dis_ref[...]
    h = jnp.maximum(dis * su + b_ref[...], 0.0)
    un = dis * jnp.dot(h, w_ref[...], preferred_element_type=jnp.float32)
    un_ref[0] = un[:, :32]
    un_ref[1] = un[:, 32:]


def _tc_layer(a, u, dis, b_prev, W_next):
    return pl.pallas_call(
        _layer_body,
        grid=(GRID,),
        in_specs=[
            pl.BlockSpec((2, BLK, 32), lambda i: (0, i, 0)),
            pl.BlockSpec((2, BLK, 32), lambda i: (0, i, 0)),
            pl.BlockSpec((BLK, 1), lambda i: (i, 0)),
            pl.BlockSpec((1, D_H), lambda i: (0, 0)),
            pl.BlockSpec((D_H, D_H), lambda i: (0, 0)),
        ],
        out_specs=pl.BlockSpec((2, BLK, 32), lambda i: (0, i, 0)),
        out_shape=jax.ShapeDtypeStruct((2, N, 32), jnp.float32),
    )(a, u, dis, b_prev, W_next)


def _final_body(a_ref, u_ref, dis_ref, b_ref, wn_ref, bn_ref, batch_ref,
                wg1_ref, bg1_ref, wg2_ref, bg2_ref, pn_ref, pg_ref,
                sums, counts):
    i = pl.program_id(0)
    su = jnp.concatenate([a_ref[0] + u_ref[0], a_ref[1] + u_ref[1]], axis=1)
    dis = dis_ref[...]
    h = jnp.maximum(dis * su + b_ref[...], 0.0)
    pn_ref[...] = (jnp.dot(h, wn_ref[...], preferred_element_type=jnp.float32)
                   + bn_ref[...])
    onehot = (batch_ref[...] ==
              lax.broadcasted_iota(jnp.int32, (BLK, G), 1)).astype(jnp.float32)
    s_inc = lax.dot_general(onehot, h, (((0,), (0,)), ((), ())),
                            preferred_element_type=jnp.float32)
    c_inc = lax.dot_general(onehot, jnp.ones((BLK, 1), jnp.float32),
                            (((0,), (0,)), ((), ())),
                            preferred_element_type=jnp.float32)

    @pl.when(i == 0)
    def _():
        sums[...] = jnp.zeros_like(sums)
        counts[...] = jnp.zeros_like(counts)

    sums[...] += s_inc
    counts[...] += c_inc

    @pl.when(i == GRID - 1)
    def _():
        emb = sums[...] / jnp.maximum(counts[...], 1.0)
        g = jnp.maximum(
            jnp.dot(emb, wg1_ref[...], preferred_element_type=jnp.float32)
            + bg1_ref[...], 0.0)
        pg_ref[...] = (jnp.dot(g, wg2_ref[...],
                               preferred_element_type=jnp.float32)
                       + bg2_ref[...])


def _tc_final(a, u, dis, b3, W_node, bn, batch2d, Wg1, bg1r, Wg2, bg2r):
    return pl.pallas_call(
        _final_body,
        grid=(GRID,),
        in_specs=[
            pl.BlockSpec((2, BLK, 32), lambda i: (0, i, 0)),
            pl.BlockSpec((2, BLK, 32), lambda i: (0, i, 0)),
            pl.BlockSpec((BLK, 1), lambda i: (i, 0)),
            pl.BlockSpec((1, D_H), lambda i: (0, 0)),
            pl.BlockSpec((D_H, D_NODE), lambda i: (0, 0)),
            pl.BlockSpec((1, D_NODE), lambda i: (0, 0)),
            pl.BlockSpec((BLK, 1), lambda i: (i, 0)),
            pl.BlockSpec((D_H, D_H), lambda i: (0, 0)),
            pl.BlockSpec((1, D_H), lambda i: (0, 0)),
            pl.BlockSpec((D_H, D_GLOBAL), lambda i: (0, 0)),
            pl.BlockSpec((1, D_GLOBAL), lambda i: (0, 0)),
        ],
        out_specs=[
            pl.BlockSpec((BLK, D_NODE), lambda i: (i, 0)),
            pl.BlockSpec((G, D_GLOBAL), lambda i: (0, 0)),
        ],
        out_shape=[
            jax.ShapeDtypeStruct((N, D_NODE), jnp.float32),
            jax.ShapeDtypeStruct((G, D_GLOBAL), jnp.float32),
        ],
        scratch_shapes=[
            pltpu.VMEM((G, D_H), jnp.float32),
            pltpu.VMEM((G, 1), jnp.float32),
        ],
    )(a, u, dis, b3, W_node, bn, batch2d, Wg1, bg1r, Wg2, bg2r)


def kernel(x, edge_index, batch, W_enc, b_enc, Wc0, bc0, Wc1, bc1, Wc2, bc2,
           Wc3, bc3, W_node, b_node, Wg1, bg1, Wg2, bg2):
    f32 = jnp.float32
    src = edge_index[0]
    dst = edge_index[1]
    pad = EPAD - E
    srcp = jnp.concatenate([src, jnp.zeros((pad,), jnp.int32)])
    dstp = jnp.concatenate(
        [dst, N + (jnp.arange(pad, dtype=jnp.int32) % DUMP)])
    s4 = srcp.reshape(16, NITER * 2, 4, 1, CHUNK)
    d4 = dstp.reshape(16, NITER * 2, 4, 1, CHUNK)
    sd_all = jnp.concatenate([
        jnp.concatenate([s4, d4], axis=3).reshape(-1),
        jnp.concatenate([s4 + N, d4], axis=3).reshape(-1)])
    dst1d = dstp
    ones8 = jnp.ones((1024, 8), f32)
    z8 = jnp.zeros((ZROWS, 8), f32)
    z32 = jnp.zeros((ZROWS, 32), f32)
    batch2d = batch.reshape(N, 1)
    be = b_enc.reshape(1, D_H)
    b0 = bc0.reshape(1, D_H)
    b1 = bc1.reshape(1, D_H)
    b2 = bc2.reshape(1, D_H)
    b3 = bc3.reshape(1, D_H)
    bn = b_node.reshape(1, D_NODE)
    bg1r = bg1.reshape(1, D_H)
    bg2r = bg2.reshape(1, D_GLOBAL)

    degp = _deg_kernel()(dst1d, ones8, z8)
    u, dis = _tc_prep(x, degp, W_enc, be, Wc0)
    a = _scatter_kernel()(u.reshape(2 * N, 32), sd_all, z32)
    u = _tc_layer(a, u, dis, b0, Wc1)
    a = _scatter_kernel()(u.reshape(2 * N, 32), sd_all, z32)
    u = _tc_layer(a, u, dis, b1, Wc2)
    a = _scatter_kernel()(u.reshape(2 * N, 32), sd_all, z32)
    u = _tc_layer(a, u, dis, b2, Wc3)
    a = _scatter_kernel()(u.reshape(2 * N, 32), sd_all, z32)
    pred_node, pred_global = _tc_final(a, u, dis, b3, W_node, bn, batch2d,
                                       Wg1, bg1r, Wg2, bg2r)
    return (pred_node, pred_global)
